# Initial kernel scaffold; baseline (speedup 1.0000x reference)
import jax
import jax.numpy as jnp
from jax.experimental import pallas as pl


def kernel(inputs, W1, b1, W2, b2, affinity_threshold):
    raise NotImplementedError("write your pallas kernel here")


# speedup vs baseline: 2.0526x; 2.0526x over previous
import functools

import jax
import jax.numpy as jnp
from jax import lax
from jax.experimental import pallas as pl
from jax.experimental.pallas import tpu as pltpu

S, H, E, D = 2048, 1024, 8, 2048
G = H // E
SCALE = E / H
WN = SCALE * (G ** 0.5)
MIN_CAP = 4
TJ = 512
TB = 256
NT = S // TB


def _gelu(v):
    return 0.5 * v * (1.0 + lax.erf(v * (2.0 ** -0.5)))


def _route_body(x_ref, thr_ref, mask_ref):
    x = x_ref[...]
    hi = lax.broadcasted_iota(jnp.int32, (H, E), 0)
    ei = lax.broadcasted_iota(jnp.int32, (H, E), 1)
    waff_t = jnp.where(hi // G == ei, jnp.float32(SCALE), jnp.float32(0.0))
    num = jnp.dot(x, waff_t)
    ssq = jnp.sum(x * x, axis=1, keepdims=True)
    den = jnp.sqrt(ssq) * WN + 1e-9
    aff = num / den
    aff_t = jnp.transpose(aff)

    mean_aff = jnp.sum(aff, keepdims=True) / (S * E)
    kf = jnp.floor(S * jax.nn.sigmoid(mean_aff - thr_ref[...]))
    kf = jnp.clip(kf, float(MIN_CAP), float(S))

    m_row = jnp.max(aff_t, axis=0, keepdims=True)
    e_iota = lax.broadcasted_iota(jnp.int32, (E, S), 0)
    top_row = jnp.min(jnp.where(aff_t == m_row, e_iota, E), axis=0,
                      keepdims=True)

    s_iota = lax.broadcasted_iota(jnp.int32, (1, S), 1)
    rows = []
    for e in range(E):
        row_vals = aff_t[e:e + 1, :]
        acc = jnp.zeros((1, S), jnp.float32)
        for j in range(S // TJ):
            col_vals = aff[j * TJ:(j + 1) * TJ, e:e + 1]
            j_iota = lax.broadcasted_iota(jnp.int32, (TJ, 1), 0) + j * TJ
            gt = (col_vals > row_vals).astype(jnp.float32)
            eqlt = ((col_vals == row_vals) & (j_iota < s_iota)).astype(jnp.float32)
            acc = acc + jnp.sum(gt + eqlt, axis=0, keepdims=True)
        ecr = acc < kf
        disp = (ecr & (top_row == e)).astype(jnp.float32)
        rows.append(disp)
    mask_rows = jnp.concatenate(rows, axis=0)
    mask_ref[...] = jnp.transpose(mask_rows)


def _route(x, thr):
    return pl.pallas_call(
        _route_body,
        out_shape=jax.ShapeDtypeStruct((S, E), jnp.float32),
    )(x, thr)


def _ffn_body(x_ref, m_ref, w1_ref, b1_ref, w2_ref, b2_ref, o_ref):
    e = pl.program_id(0)
    t = pl.program_id(1)
    sl = pl.ds(t * TB, TB)
    xb = x_ref[sl, :]
    m = m_ref[sl, :]
    ei = lax.broadcasted_iota(jnp.int32, (TB, E), 1)
    me = jnp.sum(jnp.where(ei == e, m, 0.0), axis=1, keepdims=True)
    xm = xb * me
    h = lax.dot_general(xm, w1_ref[0], (((1,), (1,)), ((), ())))
    h = _gelu(h + b1_ref[0])
    y = lax.dot_general(h, w2_ref[0], (((1,), (1,)), ((), ())))
    y = y + b2_ref[0]

    @pl.when(e == 0)
    def _init():
        o_ref[sl, :] = y

    @pl.when(e != 0)
    def _acc():
        o_ref[sl, :] = o_ref[sl, :] + y


def _ffn(x, mask, W1, b1, W2, b2):
    return pl.pallas_call(
        _ffn_body,
        grid=(E, NT),
        in_specs=[
            pl.BlockSpec((S, H), lambda e, t: (0, 0)),
            pl.BlockSpec((S, E), lambda e, t: (0, 0)),
            pl.BlockSpec((1, D, H), lambda e, t: (e, 0, 0)),
            pl.BlockSpec((1, 1, D), lambda e, t: (e, 0, 0)),
            pl.BlockSpec((1, H, D), lambda e, t: (e, 0, 0)),
            pl.BlockSpec((1, 1, H), lambda e, t: (e, 0, 0)),
        ],
        out_specs=pl.BlockSpec((S, H), lambda e, t: (0, 0)),
        out_shape=jax.ShapeDtypeStruct((S, H), jnp.float32),
        compiler_params=pltpu.CompilerParams(
            dimension_semantics=("arbitrary", "arbitrary"),
        ),
    )(x, mask, W1, b1[:, None, :], W2, b2[:, None, :])


def kernel(inputs, W1, b1, W2, b2, affinity_threshold):
    x = inputs[0]
    thr = jnp.reshape(affinity_threshold, (1, 1)).astype(jnp.float32)
    mask = _route(x, thr)
    out = _ffn(x, mask, W1, b1, W2, b2)
    return out[None]
